# baseline (speedup 1.0000x reference)
import functools

import jax
import jax.numpy as jnp
from jax import lax
from jax.experimental import pallas as pl
from jax.experimental.pallas import tpu as pltpu
from jax.experimental.pallas import tpu_sc as plsc

NUM_EXPERTS = 8
TOP_K = 2
INPUT_DIM = 10
HIDDEN = 64
OUT_DIM = 1
PAD_IN = 16
EH = NUM_EXPERTS * HIDDEN

BLOCK_T = 1024
N_TOK = 32768
NW = 32
TPW = N_TOK // NW
LANES = 16


def _router_body(lg_hbm, gates_hbm, lv, gv):
    c = lax.axis_index("c")
    s = lax.axis_index("s")
    wid = s * 2 + c
    base = wid * TPW
    pltpu.sync_copy(lg_hbm.at[:, pl.ds(base, TPW)], lv)

    def group(g, carry):
        sl = pl.ds(g * LANES, LANES)
        logit = [lv[e, sl] for e in range(NUM_EXPERTS)]
        m1 = logit[0]
        i1 = jnp.zeros((LANES,), jnp.int32)
        for e in range(1, NUM_EXPERTS):
            gt = logit[e] > m1
            m1 = jnp.where(gt, logit[e], m1)
            i1 = jnp.where(gt, e, i1)
        m2 = jnp.full((LANES,), -jnp.inf, jnp.float32)
        i2 = jnp.zeros((LANES,), jnp.int32)
        for e in range(NUM_EXPERTS):
            cand = (logit[e] > m2) & (i1 != e)
            m2 = jnp.where(cand, logit[e], m2)
            i2 = jnp.where(cand, e, i2)
        e2 = jnp.exp(m2 - m1)
        w1 = 1.0 / (1.0 + e2)
        w2 = e2 * w1
        for e in range(NUM_EXPERTS):
            gv[e, sl] = (jnp.where(i1 == e, w1, 0.0)
                         + jnp.where(i2 == e, w2, 0.0))
        return carry

    lax.fori_loop(0, TPW // LANES, group, 0)
    pltpu.sync_copy(gv, gates_hbm.at[:, pl.ds(base, TPW)])


def _sc_router(logits):
    mesh = plsc.VectorSubcoreMesh(core_axis_name="c", subcore_axis_name="s")
    return pl.kernel(
        _router_body,
        mesh=mesh,
        out_type=jax.ShapeDtypeStruct((NUM_EXPERTS, N_TOK), jnp.float32),
        scratch_types=[
            pltpu.VMEM((NUM_EXPERTS, TPW), jnp.float32),
            pltpu.VMEM((NUM_EXPERTS, TPW), jnp.float32),
        ],
    )(logits)


def _logits_block(xt_ref, wg_ref, lg_ref):
    lg_ref[...] = jnp.dot(wg_ref[...], xt_ref[...],
                          preferred_element_type=jnp.float32)


def _experts_block(xt_ref, w1_ref, w2_ref, w3_ref, eo_ref):
    xb = xt_ref[...]
    h1 = jnp.dot(w1_ref[...], xb.astype(jnp.bfloat16),
                 preferred_element_type=jnp.float32)
    h1 = jnp.maximum(h1.astype(jnp.bfloat16), jnp.bfloat16(0))
    h2 = jax.nn.relu(jnp.dot(w2_ref[...], h1,
                             preferred_element_type=jnp.float32))
    t3 = h2 * w3_ref[...]
    eo_ref[...] = jnp.sum(t3.reshape(NUM_EXPERTS, HIDDEN, -1), axis=1)


def _combine_block(g_ref, eo_ref, out_ref):
    g = g_ref[...]
    eo = eo_ref[...]
    acc = g[0:1, :] * eo[0:1, :]
    for e in range(1, NUM_EXPERTS):
        acc = acc + g[e:e + 1, :] * eo[e:e + 1, :]
    out_ref[...] = acc


@jax.jit
def kernel(x, Wg, bg, W1, b1, W2, b2, W3, b3):
    n = x.shape[0]
    xt = jnp.pad(x, ((0, 0), (0, PAD_IN - INPUT_DIM))).T
    wg_t = jnp.pad(Wg.T, ((0, 0), (0, PAD_IN - INPUT_DIM)))

    grid = (n // BLOCK_T,)
    full0 = lambda a: pl.BlockSpec(a.shape, lambda i: (0,) * a.ndim)
    logits = pl.pallas_call(
        _logits_block,
        grid=grid,
        in_specs=[
            pl.BlockSpec((PAD_IN, BLOCK_T), lambda i: (0, i)),
            full0(wg_t),
        ],
        out_specs=pl.BlockSpec((NUM_EXPERTS, BLOCK_T), lambda i: (0, i)),
        out_shape=jax.ShapeDtypeStruct((NUM_EXPERTS, n), jnp.float32),
    )(xt, wg_t)

    gates = _sc_router(logits)

    w1_t = jnp.pad(jnp.transpose(W1, (0, 2, 1)).reshape(EH, INPUT_DIM),
                   ((0, 0), (0, PAD_IN - INPUT_DIM))).astype(jnp.bfloat16)
    eye = jnp.eye(NUM_EXPERTS, dtype=jnp.float32)
    w2_t = (jnp.transpose(W2, (0, 2, 1))[:, :, None, :]
            * eye[:, None, :, None]).reshape(EH, EH).astype(jnp.bfloat16)
    w3_t = W3.reshape(EH, 1)

    full = lambda a: pl.BlockSpec(a.shape, lambda i: (0,) * a.ndim)
    eo = pl.pallas_call(
        _experts_block,
        grid=grid,
        in_specs=[
            pl.BlockSpec((PAD_IN, BLOCK_T), lambda i: (0, i)),
            full(w1_t), full(w2_t), full(w3_t),
        ],
        out_specs=pl.BlockSpec((NUM_EXPERTS, BLOCK_T), lambda i: (0, i)),
        out_shape=jax.ShapeDtypeStruct((NUM_EXPERTS, n), jnp.float32),
    )(xt, w1_t, w2_t, w3_t)

    out_t = pl.pallas_call(
        _combine_block,
        grid=grid,
        in_specs=[
            pl.BlockSpec((NUM_EXPERTS, BLOCK_T), lambda i: (0, i)),
            pl.BlockSpec((NUM_EXPERTS, BLOCK_T), lambda i: (0, i)),
        ],
        out_specs=pl.BlockSpec((1, BLOCK_T), lambda i: (0, i)),
        out_shape=jax.ShapeDtypeStruct((1, n), jnp.float32),
    )(gates, eo)
    return out_t.reshape(n, OUT_DIM)


# speedup vs baseline: 1.1461x; 1.1461x over previous
import functools

import jax
import jax.numpy as jnp
from jax import lax
from jax.experimental import pallas as pl
from jax.experimental.pallas import tpu as pltpu
from jax.experimental.pallas import tpu_sc as plsc

NUM_EXPERTS = 8
TOP_K = 2
INPUT_DIM = 10
HIDDEN = 64
OUT_DIM = 1
PAD_IN = 16
EH = NUM_EXPERTS * HIDDEN

BLOCK_T = 1024
N_TOK = 32768
NW = 32
TPW = N_TOK // NW
LANES = 16


def _router_body(lg_hbm, gates_hbm, lv, gv):
    c = lax.axis_index("c")
    s = lax.axis_index("s")
    wid = s * 2 + c
    base = wid * TPW
    pltpu.sync_copy(lg_hbm.at[:, pl.ds(base, TPW)], lv)

    def group(g, carry):
        sl = pl.ds(g * LANES, LANES)
        logit = [lv[e, sl] for e in range(NUM_EXPERTS)]
        m1 = logit[0]
        i1 = jnp.zeros((LANES,), jnp.int32)
        for e in range(1, NUM_EXPERTS):
            gt = logit[e] > m1
            m1 = jnp.where(gt, logit[e], m1)
            i1 = jnp.where(gt, e, i1)
        m2 = jnp.full((LANES,), -jnp.inf, jnp.float32)
        i2 = jnp.zeros((LANES,), jnp.int32)
        for e in range(NUM_EXPERTS):
            cand = (logit[e] > m2) & (i1 != e)
            m2 = jnp.where(cand, logit[e], m2)
            i2 = jnp.where(cand, e, i2)
        e2 = jnp.exp(m2 - m1)
        w1 = 1.0 / (1.0 + e2)
        w2 = e2 * w1
        for e in range(NUM_EXPERTS):
            gv[e, sl] = (jnp.where(i1 == e, w1, 0.0)
                         + jnp.where(i2 == e, w2, 0.0))
        return carry

    lax.fori_loop(0, TPW // LANES, group, 0)
    pltpu.sync_copy(gv, gates_hbm.at[:, pl.ds(base, TPW)])


def _sc_router(logits):
    mesh = plsc.VectorSubcoreMesh(core_axis_name="c", subcore_axis_name="s")
    return pl.kernel(
        _router_body,
        mesh=mesh,
        out_type=jax.ShapeDtypeStruct((NUM_EXPERTS, N_TOK), jnp.float32),
        scratch_types=[
            pltpu.VMEM((NUM_EXPERTS, TPW), jnp.float32),
            pltpu.VMEM((NUM_EXPERTS, TPW), jnp.float32),
        ],
    )(logits)


def _logits_block(xt_ref, wg_ref, lg_ref):
    lg_ref[...] = jnp.dot(wg_ref[...], xt_ref[...],
                          preferred_element_type=jnp.float32)


def _experts_block(xt_ref, wg_ref, w1_ref, w2_ref, w3_ref, eo_ref, lg_ref):
    xb = xt_ref[...]
    lg_ref[...] = jnp.dot(wg_ref[...], xb,
                          preferred_element_type=jnp.float32)
    h1 = jnp.dot(w1_ref[...], xb.astype(jnp.bfloat16),
                 preferred_element_type=jnp.float32)
    h1 = jnp.maximum(h1.astype(jnp.bfloat16), jnp.bfloat16(0))
    h2 = jax.nn.relu(jnp.dot(w2_ref[...], h1,
                             preferred_element_type=jnp.float32))
    t3 = h2 * w3_ref[...]
    eo_ref[...] = jnp.sum(t3.reshape(NUM_EXPERTS, HIDDEN, -1), axis=1)


def _combine_block(g_ref, eo_ref, out_ref):
    g = g_ref[...]
    eo = eo_ref[...]
    acc = g[0:1, :] * eo[0:1, :]
    for e in range(1, NUM_EXPERTS):
        acc = acc + g[e:e + 1, :] * eo[e:e + 1, :]
    out_ref[...] = acc


@jax.jit
def kernel(x, Wg, bg, W1, b1, W2, b2, W3, b3):
    n = x.shape[0]
    xt = jnp.pad(x, ((0, 0), (0, PAD_IN - INPUT_DIM))).T
    wg_t = jnp.pad(Wg.T, ((0, 0), (0, PAD_IN - INPUT_DIM)))

    grid = (n // BLOCK_T,)
    w1_t = jnp.pad(jnp.transpose(W1, (0, 2, 1)).reshape(EH, INPUT_DIM),
                   ((0, 0), (0, PAD_IN - INPUT_DIM))).astype(jnp.bfloat16)
    eye = jnp.eye(NUM_EXPERTS, dtype=jnp.float32)
    w2_t = (jnp.transpose(W2, (0, 2, 1))[:, :, None, :]
            * eye[:, None, :, None]).reshape(EH, EH).astype(jnp.bfloat16)
    w3_t = W3.reshape(EH, 1)

    full = lambda a: pl.BlockSpec(a.shape, lambda i: (0,) * a.ndim)
    eo, logits = pl.pallas_call(
        _experts_block,
        grid=grid,
        in_specs=[
            pl.BlockSpec((PAD_IN, BLOCK_T), lambda i: (0, i)),
            full(wg_t), full(w1_t), full(w2_t), full(w3_t),
        ],
        out_specs=[
            pl.BlockSpec((NUM_EXPERTS, BLOCK_T), lambda i: (0, i)),
            pl.BlockSpec((NUM_EXPERTS, BLOCK_T), lambda i: (0, i)),
        ],
        out_shape=[
            jax.ShapeDtypeStruct((NUM_EXPERTS, n), jnp.float32),
            jax.ShapeDtypeStruct((NUM_EXPERTS, n), jnp.float32),
        ],
    )(xt, wg_t, w1_t, w2_t, w3_t)

    gates = _sc_router(logits)

    out_t = pl.pallas_call(
        _combine_block,
        grid=grid,
        in_specs=[
            pl.BlockSpec((NUM_EXPERTS, BLOCK_T), lambda i: (0, i)),
            pl.BlockSpec((NUM_EXPERTS, BLOCK_T), lambda i: (0, i)),
        ],
        out_specs=pl.BlockSpec((1, BLOCK_T), lambda i: (0, i)),
        out_shape=jax.ShapeDtypeStruct((1, n), jnp.float32),
    )(gates, eo)
    return out_t.reshape(n, OUT_DIM)
